# baseline (speedup 1.0000x reference)
import functools

import jax
import jax.numpy as jnp
from jax import lax
from jax.experimental import pallas as pl
from jax.experimental.pallas import tpu as pltpu
from jax.experimental.pallas import tpu_sc as plsc

_NC = 2
_NS = 16
_LANES = 16
_CHUNK = 112

_GATHER_DNUMS = lax.GatherDimensionNumbers(
    offset_dims=(), collapsed_slice_dims=(0,), start_index_map=(0,))


def _bcast_lane(v16, e):
    idx = jnp.full((_LANES, 1), e, jnp.int32)
    return lax.gather(v16, idx, _GATHER_DNUMS, slice_sizes=(1,),
                      mode=lax.GatherScatterMode.PROMISE_IN_BOUNDS)


def _matmul_body(x_ref, w_ref, o_ref):
    o_ref[...] = jnp.dot(x_ref[...], w_ref[...], preferred_element_type=jnp.float32)


def _tc_matmul(x, w):
    m, k = x.shape
    _, n = w.shape
    bm = 1000
    return pl.pallas_call(
        _matmul_body,
        grid=(m // bm,),
        in_specs=[
            pl.BlockSpec((bm, k), lambda i: (i, 0)),
            pl.BlockSpec((k, n), lambda i: (0, 0)),
        ],
        out_specs=pl.BlockSpec((bm, n), lambda i: (i, 0)),
        out_shape=jax.ShapeDtypeStruct((m, n), jnp.float32),
    )(x, w)


def _combine_body(p_ref, b_ref, o_ref):
    o_ref[...] = p_ref[0] + p_ref[1] + b_ref[...]


def _tc_combine(partials, b1, m):
    n = partials.shape[2]
    bm = 1000
    return pl.pallas_call(
        _combine_body,
        grid=(m // bm,),
        in_specs=[
            pl.BlockSpec((2, bm, n), lambda i: (0, i, 0)),
            pl.BlockSpec((1, n), lambda i: (0, 0)),
        ],
        out_specs=pl.BlockSpec((bm, n), lambda i: (i, 0)),
        out_shape=jax.ShapeDtypeStruct((m, n), jnp.float32),
    )(partials, b1)


_GBUF = 3
_IBUF = 6


def _sc_aggregate(src_h, rows, cols, vals):
    n_nodes, d = src_h.shape
    e_pad = rows.shape[0]
    n_tiles = _NC * _NS
    per_tile = e_pad // n_tiles
    n_chunks = per_tile // _CHUNK
    assert n_chunks % _IBUF == 0 and n_chunks >= 3 * _IBUF
    n_blocks = n_chunks // _IBUF
    srows = 128
    rpt = -(-n_nodes // (_NS * srows)) * srows
    n_acc = _NS * rpt

    mesh = plsc.VectorSubcoreMesh(core_axis_name="c", subcore_axis_name="s")

    @functools.partial(
        pl.kernel,
        mesh=mesh,
        out_type=jax.ShapeDtypeStruct((_NC * n_acc, d), jnp.float32),
        scratch_types=(
            [pltpu.VMEM((_CHUNK,), jnp.int32)] * _IBUF
            + [pltpu.VMEM((_CHUNK,), jnp.int32)] * _IBUF
            + [pltpu.VMEM((_CHUNK,), jnp.float32)] * _IBUF
            + [pltpu.VMEM((_CHUNK, 128), jnp.float32)] * _GBUF
            + [pltpu.VMEM_SHARED((_NS * rpt, 128), jnp.float32)]
            + [pltpu.SemaphoreType.DMA] * (_IBUF + 2 * _GBUF)
        ),
    )
    def agg(h_hbm, rows_hbm, cols_hbm, vals_hbm, out_hbm, *refs):
        ridx_v = refs[0:_IBUF]
        cidx_v = refs[_IBUF:2 * _IBUF]
        val_v = refs[2 * _IBUF:3 * _IBUF]
        gbuf_v = refs[3 * _IBUF:3 * _IBUF + _GBUF]
        acc_sh = refs[3 * _IBUF + _GBUF]
        sems = refs[3 * _IBUF + _GBUF + 1:]
        sem_i = sems[0:_IBUF]
        sem_g = sems[_IBUF:_IBUF + _GBUF]
        sem_s = sems[_IBUF + _GBUF:]
        cid = lax.axis_index("c")
        sid = lax.axis_index("s")
        wid = sid * _NC + cid

        zvec = jnp.zeros((_LANES,), jnp.float32)

        zrows = 80

        @pl.loop(0, zrows)
        def _(i):
            for g in range(d // _LANES):
                gbuf_v[0][i, pl.ds(g * _LANES, _LANES)] = zvec

        for j in range(rpt // zrows):
            pltpu.sync_copy(
                gbuf_v[0].at[pl.ds(0, zrows)],
                acc_sh.at[pl.ds(sid * rpt + j * zrows, zrows)])

        plsc.subcore_barrier()

        base0 = wid * per_tile

        def idx_copies(c, ib):
            base = base0 + c * _CHUNK
            return (
                pltpu.make_async_copy(
                    rows_hbm.at[pl.ds(base, _CHUNK)], ridx_v[ib], sem_i[ib]),
                pltpu.make_async_copy(
                    cols_hbm.at[pl.ds(base, _CHUNK)], cidx_v[ib], sem_i[ib]),
                pltpu.make_async_copy(
                    vals_hbm.at[pl.ds(base, _CHUNK)], val_v[ib], sem_i[ib]),
            )

        def gather_copy(b, ib):
            return pltpu.make_async_copy(
                h_hbm.at[ridx_v[ib]], gbuf_v[b], sem_g[b])

        def scatter_start(b, ib):
            pltpu.async_copy(
                gbuf_v[b], acc_sh.at[cidx_v[ib]], sem_s[b], add=True)

        def scatter_wait(b, ib):
            pltpu.make_async_copy(
                gbuf_v[b], acc_sh.at[cidx_v[ib]], sem_s[b]).wait()

        def scale(b, ib):
            @pl.loop(0, _CHUNK // _LANES)
            def _(g):
                v16 = val_v[ib][pl.ds(g * _LANES, _LANES)]
                for e in range(_LANES):
                    bcast = _bcast_lane(v16, e)
                    r = g * _LANES + e
                    for f in range(d // _LANES):
                        sl = (r, pl.ds(f * _LANES, _LANES))
                        gbuf_v[b][sl] = gbuf_v[b][sl] * bcast

        def step(c, k, wait_scatter, issue_idx, issue_gather):
            b, ib = k % _GBUF, k % _IBUF
            nb, nib = (k + 1) % _GBUF, (k + 1) % _IBUF
            gather_copy(b, ib).wait()
            if wait_scatter:
                scatter_wait(nb, (k - 2) % _IBUF)
            if issue_gather:
                for cp in idx_copies(c + 1, nib):
                    cp.wait()
                gather_copy(nb, nib).start()
            if issue_idx:
                for cp in idx_copies(c + 2, (k + 2) % _IBUF):
                    cp.start()
            scale(b, ib)
            scatter_start(b, ib)

        for cp in idx_copies(0, 0):
            cp.start()
        for cp in idx_copies(1, 1):
            cp.start()
        for cp in idx_copies(0, 0):
            cp.wait()
        gather_copy(0, 0).start()

        for k in range(_IBUF):
            step(k, k, wait_scatter=k >= 2, issue_idx=True, issue_gather=True)

        @pl.loop(1, n_blocks - 1)
        def _(blk):
            c0 = blk * _IBUF
            for k in range(_IBUF):
                step(c0 + k, k, wait_scatter=True, issue_idx=True,
                     issue_gather=True)

        cl = (n_blocks - 1) * _IBUF
        for k in range(_IBUF):
            step(cl + k, k, wait_scatter=True,
                 issue_idx=k < _IBUF - 2, issue_gather=k < _IBUF - 1)
        scatter_wait((cl + _IBUF - 2) % _GBUF, _IBUF - 2)
        scatter_wait((cl + _IBUF - 1) % _GBUF, _IBUF - 1)

        plsc.subcore_barrier()

        for j in range(rpt // srows):
            r0 = sid * rpt + j * srows
            pltpu.sync_copy(acc_sh.at[pl.ds(r0, srows)],
                            out_hbm.at[pl.ds(cid * n_acc + r0, srows)])

    return agg(src_h, rows, cols, vals).reshape(_NC, n_acc, d)


def kernel(x, adj_indices, adj_values, W1, b1):
    n_nodes = x.shape[0]
    rows = adj_indices[0].astype(jnp.int32)
    cols = adj_indices[1].astype(jnp.int32)
    vals = adj_values.astype(jnp.float32)
    e = rows.shape[0]
    group = _NC * _NS * _CHUNK * _IBUF
    e_pad = ((e + group - 1) // group) * group
    if e_pad != e:
        pad = e_pad - e
        fill = jnp.arange(pad, dtype=jnp.int32) % jnp.int32(n_nodes)
        rows = jnp.concatenate([rows, fill])
        cols = jnp.concatenate([cols, fill])
        vals = jnp.concatenate([vals, jnp.zeros((pad,), jnp.float32)])

    src_h = _tc_matmul(x, W1)
    partials = _sc_aggregate(src_h, rows, cols, vals)
    return _tc_combine(partials, b1, n_nodes)


# speedup vs baseline: 1.0636x; 1.0636x over previous
import functools

import jax
import jax.numpy as jnp
from jax import lax
from jax.experimental import pallas as pl
from jax.experimental.pallas import tpu as pltpu
from jax.experimental.pallas import tpu_sc as plsc

_NC = 2
_NS = 16
_LANES = 16
_CHUNK = 80

_GATHER_DNUMS = lax.GatherDimensionNumbers(
    offset_dims=(), collapsed_slice_dims=(0,), start_index_map=(0,))


def _bcast_lane(v16, e):
    idx = jnp.full((_LANES, 1), e, jnp.int32)
    return lax.gather(v16, idx, _GATHER_DNUMS, slice_sizes=(1,),
                      mode=lax.GatherScatterMode.PROMISE_IN_BOUNDS)


def _matmul_body(x_ref, w_ref, o_ref):
    o_ref[...] = jnp.dot(x_ref[...], w_ref[...], preferred_element_type=jnp.float32)


def _tc_matmul(x, w):
    m, k = x.shape
    _, n = w.shape
    bm = 1000
    return pl.pallas_call(
        _matmul_body,
        grid=(m // bm,),
        in_specs=[
            pl.BlockSpec((bm, k), lambda i: (i, 0)),
            pl.BlockSpec((k, n), lambda i: (0, 0)),
        ],
        out_specs=pl.BlockSpec((bm, n), lambda i: (i, 0)),
        out_shape=jax.ShapeDtypeStruct((m, n), jnp.float32),
    )(x, w)


def _combine_body(p_ref, b_ref, o_ref):
    o_ref[...] = p_ref[0] + p_ref[1] + b_ref[...]


def _tc_combine(partials, b1, m):
    n = partials.shape[2]
    bm = 1000
    return pl.pallas_call(
        _combine_body,
        grid=(m // bm,),
        in_specs=[
            pl.BlockSpec((2, bm, n), lambda i: (0, i, 0)),
            pl.BlockSpec((1, n), lambda i: (0, 0)),
        ],
        out_specs=pl.BlockSpec((bm, n), lambda i: (i, 0)),
        out_shape=jax.ShapeDtypeStruct((m, n), jnp.float32),
    )(partials, b1)


_GBUF = 4
_IBUF = 8


def _sc_aggregate(src_h, rows, cols, vals):
    n_nodes, d = src_h.shape
    e_pad = rows.shape[0]
    n_tiles = _NC * _NS
    per_tile = e_pad // n_tiles
    n_chunks = per_tile // _CHUNK
    assert n_chunks % _IBUF == 0 and n_chunks >= 3 * _IBUF
    n_blocks = n_chunks // _IBUF
    srows = 128
    rpt = -(-n_nodes // (_NS * srows)) * srows
    n_acc = _NS * rpt

    mesh = plsc.VectorSubcoreMesh(core_axis_name="c", subcore_axis_name="s")

    @functools.partial(
        pl.kernel,
        mesh=mesh,
        out_type=jax.ShapeDtypeStruct((_NC * n_acc, d), jnp.float32),
        scratch_types=(
            [pltpu.VMEM((_CHUNK,), jnp.int32)] * _IBUF
            + [pltpu.VMEM((_CHUNK,), jnp.int32)] * _IBUF
            + [pltpu.VMEM((_CHUNK,), jnp.float32)] * _IBUF
            + [pltpu.VMEM((_CHUNK, 128), jnp.float32)] * _GBUF
            + [pltpu.VMEM_SHARED((_NS * rpt, 128), jnp.float32)]
            + [pltpu.SemaphoreType.DMA] * (_IBUF + 2 * _GBUF)
        ),
    )
    def agg(h_hbm, rows_hbm, cols_hbm, vals_hbm, out_hbm, *refs):
        ridx_v = refs[0:_IBUF]
        cidx_v = refs[_IBUF:2 * _IBUF]
        val_v = refs[2 * _IBUF:3 * _IBUF]
        gbuf_v = refs[3 * _IBUF:3 * _IBUF + _GBUF]
        acc_sh = refs[3 * _IBUF + _GBUF]
        sems = refs[3 * _IBUF + _GBUF + 1:]
        sem_i = sems[0:_IBUF]
        sem_g = sems[_IBUF:_IBUF + _GBUF]
        sem_s = sems[_IBUF + _GBUF:]
        cid = lax.axis_index("c")
        sid = lax.axis_index("s")
        wid = sid * _NC + cid

        zvec = jnp.zeros((_LANES,), jnp.float32)

        zrows = 80

        @pl.loop(0, zrows)
        def _(i):
            for g in range(d // _LANES):
                gbuf_v[0][i, pl.ds(g * _LANES, _LANES)] = zvec

        for j in range(rpt // zrows):
            pltpu.sync_copy(
                gbuf_v[0].at[pl.ds(0, zrows)],
                acc_sh.at[pl.ds(sid * rpt + j * zrows, zrows)])

        plsc.subcore_barrier()

        base0 = wid * per_tile

        def idx_copies(c, ib):
            base = base0 + c * _CHUNK
            return (
                pltpu.make_async_copy(
                    rows_hbm.at[pl.ds(base, _CHUNK)], ridx_v[ib], sem_i[ib]),
                pltpu.make_async_copy(
                    cols_hbm.at[pl.ds(base, _CHUNK)], cidx_v[ib], sem_i[ib]),
                pltpu.make_async_copy(
                    vals_hbm.at[pl.ds(base, _CHUNK)], val_v[ib], sem_i[ib]),
            )

        def gather_copy(b, ib):
            return pltpu.make_async_copy(
                h_hbm.at[ridx_v[ib]], gbuf_v[b], sem_g[b])

        def scatter_start(b, ib):
            pltpu.async_copy(
                gbuf_v[b], acc_sh.at[cidx_v[ib]], sem_s[b], add=True)

        def scatter_wait(b, ib):
            pltpu.make_async_copy(
                gbuf_v[b], acc_sh.at[cidx_v[ib]], sem_s[b]).wait()

        def scale(b, ib):
            @pl.loop(0, _CHUNK // _LANES)
            def _(g):
                v16 = val_v[ib][pl.ds(g * _LANES, _LANES)]
                for e in range(_LANES):
                    bcast = _bcast_lane(v16, e)
                    r = g * _LANES + e
                    for f in range(d // _LANES):
                        sl = (r, pl.ds(f * _LANES, _LANES))
                        gbuf_v[b][sl] = gbuf_v[b][sl] * bcast

        def step(c, k, wait_scatter, issue_idx, issue_gather):
            b, ib = k % _GBUF, k % _IBUF
            gather_copy(b, ib).wait()
            if wait_scatter:
                scatter_wait((k - 1) % _GBUF, (k - 1) % _IBUF)
            if issue_gather:
                for cp in idx_copies(c + 3, (k + 3) % _IBUF):
                    cp.wait()
                gather_copy((k + 3) % _GBUF, (k + 3) % _IBUF).start()
            if issue_idx:
                for cp in idx_copies(c + 4, (k + 4) % _IBUF):
                    cp.start()
            scale(b, ib)
            scatter_start(b, ib)

        for j in range(4):
            for cp in idx_copies(j, j):
                cp.start()
        for j in range(3):
            for cp in idx_copies(j, j):
                cp.wait()
            gather_copy(j, j).start()

        for k in range(_IBUF):
            step(k, k, wait_scatter=k >= 1, issue_idx=True, issue_gather=True)

        @pl.loop(1, n_blocks - 1)
        def _(blk):
            c0 = blk * _IBUF
            for k in range(_IBUF):
                step(c0 + k, k, wait_scatter=True, issue_idx=True,
                     issue_gather=True)

        cl = (n_blocks - 1) * _IBUF
        for k in range(_IBUF):
            step(cl + k, k, wait_scatter=True,
                 issue_idx=k < _IBUF - 4, issue_gather=k < _IBUF - 3)
        scatter_wait((_IBUF - 1) % _GBUF, _IBUF - 1)

        plsc.subcore_barrier()

        for j in range(rpt // srows):
            r0 = sid * rpt + j * srows
            pltpu.sync_copy(acc_sh.at[pl.ds(r0, srows)],
                            out_hbm.at[pl.ds(cid * n_acc + r0, srows)])

    return agg(src_h, rows, cols, vals).reshape(_NC, n_acc, d)


def kernel(x, adj_indices, adj_values, W1, b1):
    n_nodes = x.shape[0]
    rows = adj_indices[0].astype(jnp.int32)
    cols = adj_indices[1].astype(jnp.int32)
    vals = adj_values.astype(jnp.float32)
    e = rows.shape[0]
    group = _NC * _NS * _CHUNK * _IBUF
    e_pad = ((e + group - 1) // group) * group
    if e_pad != e:
        pad = e_pad - e
        fill = jnp.arange(pad, dtype=jnp.int32) % jnp.int32(n_nodes)
        rows = jnp.concatenate([rows, fill])
        cols = jnp.concatenate([cols, fill])
        vals = jnp.concatenate([vals, jnp.zeros((pad,), jnp.float32)])

    src_h = _tc_matmul(x, W1)
    partials = _sc_aggregate(src_h, rows, cols, vals)
    return _tc_combine(partials, b1, n_nodes)
